# baseline (speedup 1.0000x reference)
import jax
import jax.numpy as jnp
from jax.experimental import pallas as pl
from jax.experimental.pallas import tpu as pltpu

NBINS = 100
NCHUNK = 4


def _norm_body(t_ref, mean_ref, std_ref, x_hbm, o_hbm, buf, obuf, m_sc,
               inv_sc, rsems, wsems):
    F = x_hbm.shape[0]
    S = F // NCHUNK
    for k in range(NCHUNK):
        pltpu.make_async_copy(
            x_hbm.at[pl.ds(k * S, S), :], buf.at[k], rsems.at[k]
        ).start()

    tr = t_ref[...]
    bins = jnp.clip((tr * NBINS).astype(jnp.int32), 0, NBINS - 1)
    krows = jax.lax.broadcasted_iota(jnp.int32, (NBINS, 1), 0)
    oh = (krows == bins).astype(jnp.float32)
    m_sc[...] = jnp.dot(
        mean_ref[...], oh, preferred_element_type=jnp.float32,
        precision=jax.lax.Precision.HIGHEST,
    )
    s = jnp.dot(
        std_ref[...], oh, preferred_element_type=jnp.float32,
        precision=jax.lax.Precision.HIGHEST,
    )
    inv_sc[...] = 1.0 / s

    for k in range(NCHUNK):
        pltpu.make_async_copy(
            x_hbm.at[pl.ds(k * S, S), :], buf.at[k], rsems.at[k]
        ).wait()
        obuf[k] = (buf[k] - m_sc[...]) * inv_sc[...]
        pltpu.make_async_copy(
            obuf.at[k], o_hbm.at[pl.ds(k * S, S), :], wsems.at[k]
        ).start()
    for k in range(NCHUNK):
        pltpu.make_async_copy(
            obuf.at[k], o_hbm.at[pl.ds(k * S, S), :], wsems.at[k]
        ).wait()


def kernel(x_t, t, data_mean, data_std):
    B, C, H, W = x_t.shape
    F = C * H * W
    xv = jnp.transpose(x_t, (1, 2, 3, 0)).reshape(F, B)
    xv = pltpu.with_memory_space_constraint(xv, pltpu.HBM)
    t_row = t.reshape(1, B)
    mean_row = data_mean.reshape(1, NBINS)
    std_row = data_std.reshape(1, NBINS)

    S = F // NCHUNK
    out = pl.pallas_call(
        _norm_body,
        in_specs=[
            pl.BlockSpec((1, B), lambda: (0, 0)),
            pl.BlockSpec((1, NBINS), lambda: (0, 0)),
            pl.BlockSpec((1, NBINS), lambda: (0, 0)),
            pl.BlockSpec(memory_space=pltpu.HBM),
        ],
        out_specs=pl.BlockSpec(memory_space=pltpu.HBM),
        out_shape=jax.ShapeDtypeStruct((F, B), jnp.float32),
        scratch_shapes=[
            pltpu.VMEM((NCHUNK, S, B), jnp.float32),
            pltpu.VMEM((NCHUNK, S, B), jnp.float32),
            pltpu.VMEM((1, B), jnp.float32),
            pltpu.VMEM((1, B), jnp.float32),
            pltpu.SemaphoreType.DMA((NCHUNK,)),
            pltpu.SemaphoreType.DMA((NCHUNK,)),
        ],
    )(t_row, mean_row, std_row, xv)
    return jnp.transpose(out.reshape(C, H, W, B), (3, 0, 1, 2))
